# baseline (speedup 1.0000x reference)
import jax
import jax.numpy as jnp
from jax.experimental import pallas as pl
from jax.experimental.pallas import tpu as pltpu

NUM_Q = 100000
D = 256
BLOCK = 1000
GRID = NUM_Q // BLOCK


def _copy_body(e_in, p_in, e_out, p_out):
    e_out[...] = e_in[...]
    p_out[...] = p_in[...]


def kernel(batch_size, query_embed_weight, query_pos_weight):
    out = jax.ShapeDtypeStruct((NUM_Q, D), jnp.float32)
    spec = pl.BlockSpec((BLOCK, D), lambda i: (i, 0))
    query_embed, query_pos = pl.pallas_call(
        _copy_body,
        grid=(GRID,),
        in_specs=[spec, spec],
        out_specs=[spec, spec],
        out_shape=[out, out],
    )(query_embed_weight, query_pos_weight)
    return (query_embed, query_pos)


# speedup vs baseline: 1.1201x; 1.1201x over previous
import jax
import jax.numpy as jnp
from jax.experimental import pallas as pl
from jax.experimental.pallas import tpu as pltpu

NUM_Q = 100000
D = 256
BLOCK = 5000
GRID = NUM_Q // BLOCK


def _copy_body(e_in, p_in, e_out, p_out):
    e_out[...] = e_in[...]
    p_out[...] = p_in[...]


def kernel(batch_size, query_embed_weight, query_pos_weight):
    out = jax.ShapeDtypeStruct((NUM_Q, D), jnp.float32)
    spec = pl.BlockSpec((BLOCK, D), lambda i: (i, 0))
    query_embed, query_pos = pl.pallas_call(
        _copy_body,
        grid=(GRID,),
        in_specs=[spec, spec],
        out_specs=[spec, spec],
        out_shape=[out, out],
    )(query_embed_weight, query_pos_weight)
    return (query_embed, query_pos)
